# baseline (speedup 1.0000x reference)
import functools

import jax
import jax.numpy as jnp
from jax import lax
from jax.experimental import pallas as pl
from jax.experimental.pallas import tpu as pltpu
from jax.experimental.pallas import tpu_sc as plsc

HIDDEN = 64
EPS = 1e-12
CHUNK = 128
NC, NS = 2, 16
NW = NC * NS
TC_ROWS = 4096


@functools.lru_cache(maxsize=None)
def _build_gather_sum(n_rows):
    assert n_rows % (NW * CHUNK) == 0
    per_w = n_rows // NW
    n_chunks = per_w // CHUNK
    mesh = plsc.VectorSubcoreMesh(
        core_axis_name="c", subcore_axis_name="s", num_cores=NC, num_subcores=NS
    )

    @functools.partial(
        pl.kernel,
        out_type=jax.ShapeDtypeStruct((n_rows, HIDDEN), jnp.float32),
        mesh=mesh,
        scratch_types=dict(
            idx_d=pltpu.VMEM((2, CHUNK), jnp.int32),
            idx_m=pltpu.VMEM((2, CHUNK), jnp.int32),
            buf=pltpu.VMEM((2, CHUNK, HIDDEN), jnp.float32),
            sem_d=pltpu.SemaphoreType.DMA((2,)),
            sem_m=pltpu.SemaphoreType.DMA((2,)),
            sem_o=pltpu.SemaphoreType.DMA((2,)),
        ),
        compiler_params=pltpu.CompilerParams(
            needs_layout_passes=False, use_tc_tiling_on_sc=False
        ),
    )
    def sc_kernel(diag_hbm, med_hbm, wd_hbm, wm_hbm, out_hbm, *,
                  idx_d, idx_m, buf, sem_d, sem_m, sem_o):
        wid = lax.axis_index("s") * NC + lax.axis_index("c")
        base = wid * per_w

        def fetch(ci, b):
            rb = base + ci * CHUNK
            pltpu.sync_copy(diag_hbm.at[pl.ds(rb, CHUNK)], idx_d.at[b])
            pltpu.sync_copy(med_hbm.at[pl.ds(rb, CHUNK)], idx_m.at[b])
            pltpu.async_copy(wd_hbm.at[idx_d.at[b]], buf.at[b], sem_d.at[b])

        def wait_gather(b):
            pltpu.make_async_copy(wd_hbm.at[idx_d.at[b]], buf.at[b],
                                  sem_d.at[b]).wait()

        def wait_out(ci, b):
            rb = base + ci * CHUNK
            pltpu.make_async_copy(buf.at[b], out_hbm.at[pl.ds(rb, CHUNK)],
                                  sem_o.at[b]).wait()

        fetch(0, 0)

        def outer(i, carry):
            for k in (0, 1):
                ci = 2 * i + k
                b = k
                nb = 1 - k

                @pl.when(ci + 1 < n_chunks)
                def _():
                    @pl.when(ci >= 1)
                    def _():
                        wait_out(ci - 1, nb)
                    fetch(ci + 1, nb)

                wait_gather(b)
                cm = pltpu.async_copy(wm_hbm.at[idx_m.at[b]], buf.at[b],
                                      sem_m.at[b], add=True)
                cm.wait()
                rb = base + ci * CHUNK
                pltpu.async_copy(buf.at[b], out_hbm.at[pl.ds(rb, CHUNK)],
                                 sem_o.at[b])
            return carry

        lax.fori_loop(0, n_chunks // 2, outer, 0)
        wait_out(n_chunks - 2, 0)
        wait_out(n_chunks - 1, 1)

    return sc_kernel


@functools.lru_cache(maxsize=None)
def _build_layernorm(B, L):
    BB = 128
    R = BB * L // 2
    n_flat = B * L * HIDDEN

    def body(x_ref, gg_ref, bb_ref, o_ref):
        x = x_ref[...]
        half = lax.broadcasted_iota(jnp.int32, (2 * HIDDEN, 2 * HIDDEN), 0) // HIDDEN
        halfc = lax.broadcasted_iota(jnp.int32, (2 * HIDDEN, 2 * HIDDEN), 1) // HIDDEN
        S = jnp.where(half == halfc, 1.0 / HIDDEN, 0.0).astype(jnp.float32)
        mean = jnp.dot(x, S, preferred_element_type=jnp.float32)
        meansq = jnp.dot(x * x, S, preferred_element_type=jnp.float32)
        var = meansq - mean * mean
        y = (x - mean) * lax.rsqrt(var + EPS) * gg_ref[...] + bb_ref[...]
        o_ref[...] = y.reshape(BB, L // 2, 2 * HIDDEN)

    return pl.pallas_call(
        body,
        grid=(B // BB,),
        in_specs=[
            pl.BlockSpec((R, 2 * HIDDEN), lambda i: (i, 0)),
            pl.BlockSpec((1, 2 * HIDDEN), lambda i: (0, 0)),
            pl.BlockSpec((1, 2 * HIDDEN), lambda i: (0, 0)),
        ],
        out_specs=pl.BlockSpec((BB, L // 2, 2 * HIDDEN), lambda i: (i, 0, 0)),
        out_shape=jax.ShapeDtypeStruct((B, L // 2, 2 * HIDDEN), jnp.float32),
    )


NPART = 4


def kernel(diag_ids, med_ids, W_diag, W_med, gamma, beta):
    B, L = diag_ids.shape
    Bk = B // NPART
    nk = Bk * L
    g2 = jnp.concatenate([gamma, gamma]).reshape(1, 2 * HIDDEN)
    b2 = jnp.concatenate([beta, beta]).reshape(1, 2 * HIDDEN)
    dflat = diag_ids.astype(jnp.int32).reshape(B * L)
    mflat = med_ids.astype(jnp.int32).reshape(B * L)
    sc = _build_gather_sum(nk)
    tc = _build_layernorm(Bk, L)
    parts = []
    for k in range(NPART):
        summed = sc(
            lax.dynamic_slice_in_dim(dflat, k * nk, nk),
            lax.dynamic_slice_in_dim(mflat, k * nk, nk),
            W_diag, W_med,
        )
        parts.append(tc(summed.reshape(nk // 2, 2 * HIDDEN), g2, b2))
    return jnp.concatenate(parts, axis=0).reshape(B, L, HIDDEN)


# speedup vs baseline: 1.0462x; 1.0462x over previous
import functools

import jax
import jax.numpy as jnp
from jax import lax
from jax.experimental import pallas as pl
from jax.experimental.pallas import tpu as pltpu
from jax.experimental.pallas import tpu_sc as plsc

HIDDEN = 64
EPS = 1e-12
CHUNK = 128
NC, NS = 2, 16
NW = NC * NS
TC_ROWS = 4096


@functools.lru_cache(maxsize=None)
def _build_gather_sum(n_rows):
    assert n_rows % (NW * CHUNK) == 0
    per_w = n_rows // NW
    n_chunks = per_w // CHUNK
    mesh = plsc.VectorSubcoreMesh(
        core_axis_name="c", subcore_axis_name="s", num_cores=NC, num_subcores=NS
    )

    @functools.partial(
        pl.kernel,
        out_type=jax.ShapeDtypeStruct((n_rows, HIDDEN), jnp.float32),
        mesh=mesh,
        scratch_types=dict(
            idx_d=pltpu.VMEM((2, CHUNK), jnp.int32),
            idx_m=pltpu.VMEM((2, CHUNK), jnp.int32),
            buf=pltpu.VMEM((2, CHUNK, HIDDEN), jnp.float32),
            sem_d=pltpu.SemaphoreType.DMA((2,)),
            sem_m=pltpu.SemaphoreType.DMA((2,)),
            sem_o=pltpu.SemaphoreType.DMA((2,)),
        ),
        compiler_params=pltpu.CompilerParams(
            needs_layout_passes=False, use_tc_tiling_on_sc=False
        ),
    )
    def sc_kernel(diag_hbm, med_hbm, wd_hbm, wm_hbm, out_hbm, *,
                  idx_d, idx_m, buf, sem_d, sem_m, sem_o):
        wid = lax.axis_index("s") * NC + lax.axis_index("c")
        base = wid * per_w

        def fetch(ci, b):
            rb = base + ci * CHUNK
            pltpu.sync_copy(diag_hbm.at[pl.ds(rb, CHUNK)], idx_d.at[b])
            pltpu.sync_copy(med_hbm.at[pl.ds(rb, CHUNK)], idx_m.at[b])
            pltpu.async_copy(wd_hbm.at[idx_d.at[b]], buf.at[b], sem_d.at[b])

        def wait_gather(b):
            pltpu.make_async_copy(wd_hbm.at[idx_d.at[b]], buf.at[b],
                                  sem_d.at[b]).wait()

        def wait_out(ci, b):
            rb = base + ci * CHUNK
            pltpu.make_async_copy(buf.at[b], out_hbm.at[pl.ds(rb, CHUNK)],
                                  sem_o.at[b]).wait()

        fetch(0, 0)

        def outer(i, carry):
            for k in (0, 1):
                ci = 2 * i + k
                b = k
                nb = 1 - k

                @pl.when(ci + 1 < n_chunks)
                def _():
                    @pl.when(ci >= 1)
                    def _():
                        wait_out(ci - 1, nb)
                    fetch(ci + 1, nb)

                wait_gather(b)
                cm = pltpu.async_copy(wm_hbm.at[idx_m.at[b]], buf.at[b],
                                      sem_m.at[b], add=True)
                cm.wait()
                rb = base + ci * CHUNK
                pltpu.async_copy(buf.at[b], out_hbm.at[pl.ds(rb, CHUNK)],
                                 sem_o.at[b])
            return carry

        lax.fori_loop(0, n_chunks // 2, outer, 0)
        wait_out(n_chunks - 2, 0)
        wait_out(n_chunks - 1, 1)

    return sc_kernel


@functools.lru_cache(maxsize=None)
def _build_layernorm(B, L):
    BB = 128
    R = BB * L // 2
    n_flat = B * L * HIDDEN

    def body(x_ref, gg_ref, bb_ref, o_ref):
        x = x_ref[...]
        half = lax.broadcasted_iota(jnp.int32, (2 * HIDDEN, 2 * HIDDEN), 0) // HIDDEN
        halfc = lax.broadcasted_iota(jnp.int32, (2 * HIDDEN, 2 * HIDDEN), 1) // HIDDEN
        S = jnp.where(half == halfc, 1.0 / HIDDEN, 0.0).astype(jnp.float32)
        mean = jnp.dot(x, S, preferred_element_type=jnp.float32)
        meansq = jnp.dot(x * x, S, preferred_element_type=jnp.float32)
        var = meansq - mean * mean
        y = (x - mean) * lax.rsqrt(var + EPS) * gg_ref[...] + bb_ref[...]
        o_ref[...] = y.reshape(BB, L // 2, 2 * HIDDEN)

    return pl.pallas_call(
        body,
        grid=(B // BB,),
        in_specs=[
            pl.BlockSpec((R, 2 * HIDDEN), lambda i: (i, 0)),
            pl.BlockSpec((1, 2 * HIDDEN), lambda i: (0, 0)),
            pl.BlockSpec((1, 2 * HIDDEN), lambda i: (0, 0)),
        ],
        out_specs=pl.BlockSpec((BB, L // 2, 2 * HIDDEN), lambda i: (i, 0, 0)),
        out_shape=jax.ShapeDtypeStruct((B, L // 2, 2 * HIDDEN), jnp.float32),
    )


NPART = 4


def kernel(diag_ids, med_ids, W_diag, W_med, gamma, beta):
    B, L = diag_ids.shape
    Bk = B // NPART
    nk = Bk * L
    g2 = jnp.concatenate([gamma, gamma]).reshape(1, 2 * HIDDEN)
    b2 = jnp.concatenate([beta, beta]).reshape(1, 2 * HIDDEN)
    dflat = diag_ids.astype(jnp.int32).reshape(B * L)
    mflat = med_ids.astype(jnp.int32).reshape(B * L)
    sc = _build_gather_sum(nk)
    tc = _build_layernorm(Bk, L)
    out = jnp.zeros((B, L, HIDDEN), jnp.float32)
    for k in range(NPART):
        summed = sc(
            lax.dynamic_slice_in_dim(dflat, k * nk, nk),
            lax.dynamic_slice_in_dim(mflat, k * nk, nk),
            W_diag, W_med,
        )
        part = tc(summed.reshape(nk // 2, 2 * HIDDEN), g2, b2)
        out = lax.dynamic_update_slice(
            out, part.reshape(Bk, L, HIDDEN), (k * Bk, 0, 0)
        )
    return out


# speedup vs baseline: 1.1063x; 1.0574x over previous
import functools

import jax
import jax.numpy as jnp
from jax import lax
from jax.experimental import pallas as pl
from jax.experimental.pallas import tpu as pltpu
from jax.experimental.pallas import tpu_sc as plsc

HIDDEN = 64
EPS = 1e-12
CHUNK = 128
NC, NS = 2, 16
NW = NC * NS
TC_ROWS = 4096


@functools.lru_cache(maxsize=None)
def _build_gather_sum(n_rows):
    assert n_rows % (NW * CHUNK) == 0
    per_w = n_rows // NW
    n_chunks = per_w // CHUNK
    mesh = plsc.VectorSubcoreMesh(
        core_axis_name="c", subcore_axis_name="s", num_cores=NC, num_subcores=NS
    )

    @functools.partial(
        pl.kernel,
        out_type=jax.ShapeDtypeStruct((n_rows, HIDDEN), jnp.float32),
        mesh=mesh,
        scratch_types=dict(
            idx_d=pltpu.VMEM((2, CHUNK), jnp.int32),
            idx_m=pltpu.VMEM((2, CHUNK), jnp.int32),
            buf=pltpu.VMEM((2, CHUNK, HIDDEN), jnp.float32),
            sem_d=pltpu.SemaphoreType.DMA((2,)),
            sem_m=pltpu.SemaphoreType.DMA((2,)),
            sem_o=pltpu.SemaphoreType.DMA((2,)),
        ),
        compiler_params=pltpu.CompilerParams(
            needs_layout_passes=False, use_tc_tiling_on_sc=False
        ),
    )
    def sc_kernel(diag_hbm, med_hbm, wd_hbm, wm_hbm, out_hbm, *,
                  idx_d, idx_m, buf, sem_d, sem_m, sem_o):
        wid = lax.axis_index("s") * NC + lax.axis_index("c")
        base = wid * per_w

        def fetch(ci, b):
            rb = base + ci * CHUNK
            pltpu.sync_copy(diag_hbm.at[pl.ds(rb, CHUNK)], idx_d.at[b])
            pltpu.sync_copy(med_hbm.at[pl.ds(rb, CHUNK)], idx_m.at[b])
            pltpu.async_copy(wd_hbm.at[idx_d.at[b]], buf.at[b], sem_d.at[b])

        def wait_gather(b):
            pltpu.make_async_copy(wd_hbm.at[idx_d.at[b]], buf.at[b],
                                  sem_d.at[b]).wait()

        def wait_out(ci, b):
            rb = base + ci * CHUNK
            pltpu.make_async_copy(buf.at[b], out_hbm.at[pl.ds(rb, CHUNK)],
                                  sem_o.at[b]).wait()

        fetch(0, 0)

        def outer(i, carry):
            for k in (0, 1):
                ci = 2 * i + k
                b = k
                nb = 1 - k

                @pl.when(ci + 1 < n_chunks)
                def _():
                    @pl.when(ci >= 1)
                    def _():
                        wait_out(ci - 1, nb)
                    fetch(ci + 1, nb)

                wait_gather(b)
                cm = pltpu.async_copy(wm_hbm.at[idx_m.at[b]], buf.at[b],
                                      sem_m.at[b], add=True)
                cm.wait()
                rb = base + ci * CHUNK
                pltpu.async_copy(buf.at[b], out_hbm.at[pl.ds(rb, CHUNK)],
                                 sem_o.at[b])
            return carry

        lax.fori_loop(0, n_chunks // 2, outer, 0)
        wait_out(n_chunks - 2, 0)
        wait_out(n_chunks - 1, 1)

    return sc_kernel


@functools.lru_cache(maxsize=None)
def _build_layernorm(B, L):
    BB = 128
    R = BB * L // 2
    n_flat = B * L * HIDDEN

    def body(x_ref, gg_ref, bb_ref, o_ref):
        x = x_ref[...]
        half = lax.broadcasted_iota(jnp.int32, (2 * HIDDEN, 2 * HIDDEN), 0) // HIDDEN
        halfc = lax.broadcasted_iota(jnp.int32, (2 * HIDDEN, 2 * HIDDEN), 1) // HIDDEN
        S = jnp.where(half == halfc, 1.0 / HIDDEN, 0.0).astype(jnp.float32)
        mean = jnp.dot(x, S, preferred_element_type=jnp.float32)
        meansq = jnp.dot(x * x, S, preferred_element_type=jnp.float32)
        var = meansq - mean * mean
        y = (x - mean) * lax.rsqrt(var + EPS) * gg_ref[...] + bb_ref[...]
        o_ref[...] = y.reshape(BB, L // 2, 2 * HIDDEN)

    return pl.pallas_call(
        body,
        grid=(B // BB,),
        in_specs=[
            pl.BlockSpec((R, 2 * HIDDEN), lambda i: (i, 0)),
            pl.BlockSpec((1, 2 * HIDDEN), lambda i: (0, 0)),
            pl.BlockSpec((1, 2 * HIDDEN), lambda i: (0, 0)),
        ],
        out_specs=pl.BlockSpec((BB, L // 2, 2 * HIDDEN), lambda i: (i, 0, 0)),
        out_shape=jax.ShapeDtypeStruct((B, L // 2, 2 * HIDDEN), jnp.float32),
    )


NPART = 4


def kernel(diag_ids, med_ids, W_diag, W_med, gamma, beta):
    B, L = diag_ids.shape
    Bk = B // NPART
    nk = Bk * L
    g2 = jnp.concatenate([gamma, gamma]).reshape(1, 2 * HIDDEN)
    b2 = jnp.concatenate([beta, beta]).reshape(1, 2 * HIDDEN)
    d32 = diag_ids.astype(jnp.int32)
    m32 = med_ids.astype(jnp.int32)
    sc = _build_gather_sum(nk)
    tc = _build_layernorm(Bk, L)
    parts = []
    for k in range(NPART):
        summed = sc(
            lax.dynamic_slice_in_dim(d32, k * Bk, Bk).reshape(nk),
            lax.dynamic_slice_in_dim(m32, k * Bk, Bk).reshape(nk),
            W_diag, W_med,
        )
        part = tc(summed.reshape(nk // 2, 2 * HIDDEN), g2, b2)
        parts.append(part.reshape(Bk, L, HIDDEN))
    return jnp.concatenate(parts, axis=0)


# speedup vs baseline: 1.3641x; 1.2331x over previous
import functools

import jax
import jax.numpy as jnp
from jax import lax
from jax.experimental import pallas as pl
from jax.experimental.pallas import tpu as pltpu
from jax.experimental.pallas import tpu_sc as plsc

HIDDEN = 64
EPS = 1e-12
CHUNK = 128
NC, NS = 2, 16
NW = NC * NS
TC_ROWS = 4096


@functools.lru_cache(maxsize=None)
def _build_gather_sum(n_rows):
    assert n_rows % (NW * CHUNK) == 0
    per_w = n_rows // NW
    n_chunks = per_w // CHUNK
    mesh = plsc.VectorSubcoreMesh(
        core_axis_name="c", subcore_axis_name="s", num_cores=NC, num_subcores=NS
    )

    NBUF = 5
    assert n_chunks % NBUF == 0 and n_chunks >= 2 * NBUF

    @functools.partial(
        pl.kernel,
        out_type=jax.ShapeDtypeStruct((n_rows, HIDDEN), jnp.float32),
        mesh=mesh,
        scratch_types=dict(
            idx_d=pltpu.VMEM((NBUF, CHUNK), jnp.int32),
            idx_m=pltpu.VMEM((NBUF, CHUNK), jnp.int32),
            buf=pltpu.VMEM((NBUF, CHUNK, HIDDEN), jnp.float32),
            sem_d=pltpu.SemaphoreType.DMA((NBUF,)),
            sem_m=pltpu.SemaphoreType.DMA((NBUF,)),
            sem_o=pltpu.SemaphoreType.DMA((NBUF,)),
        ),
        compiler_params=pltpu.CompilerParams(
            needs_layout_passes=False, use_tc_tiling_on_sc=False
        ),
    )
    def sc_kernel(diag_hbm, med_hbm, wd_hbm, wm_hbm, out_hbm, *,
                  idx_d, idx_m, buf, sem_d, sem_m, sem_o):
        wid = lax.axis_index("s") * NC + lax.axis_index("c")
        base = wid * per_w

        def fetch(ci, b):
            rb = base + ci * CHUNK
            pltpu.sync_copy(diag_hbm.at[pl.ds(rb, CHUNK)], idx_d.at[b])
            pltpu.sync_copy(med_hbm.at[pl.ds(rb, CHUNK)], idx_m.at[b])
            pltpu.async_copy(wd_hbm.at[idx_d.at[b]], buf.at[b], sem_d.at[b])

        def wait_diag(b):
            pltpu.make_async_copy(wd_hbm.at[idx_d.at[b]], buf.at[b],
                                  sem_d.at[b]).wait()

        def fire_med(b):
            pltpu.async_copy(wm_hbm.at[idx_m.at[b]], buf.at[b],
                             sem_m.at[b], add=True)

        def wait_med(b):
            pltpu.make_async_copy(wm_hbm.at[idx_m.at[b]], buf.at[b],
                                  sem_m.at[b]).wait()

        def fire_out(ci, b):
            rb = base + ci * CHUNK
            pltpu.async_copy(buf.at[b], out_hbm.at[pl.ds(rb, CHUNK)],
                             sem_o.at[b])

        def wait_out(ci, b):
            rb = base + ci * CHUNK
            pltpu.make_async_copy(buf.at[b], out_hbm.at[pl.ds(rb, CHUNK)],
                                  sem_o.at[b]).wait()

        fetch(0, 0)

        def outer(i, carry):
            for k in range(NBUF):
                ci = NBUF * i + k

                @pl.when(ci >= 4)
                def _():
                    wait_out(ci - 4, (ci - 4) % NBUF)

                @pl.when(ci + 1 < n_chunks)
                def _():
                    fetch(ci + 1, (ci + 1) % NBUF)

                wait_diag(k)
                fire_med(k)

                @pl.when(ci >= 1)
                def _():
                    bp = (ci - 1) % NBUF
                    wait_med(bp)
                    fire_out(ci - 1, bp)
            return carry

        lax.fori_loop(0, n_chunks // NBUF, outer, 0)
        last = n_chunks - 1
        wait_med(last % NBUF)
        fire_out(last, last % NBUF)
        for ci in range(n_chunks - 4, n_chunks):
            wait_out(ci, ci % NBUF)

    return sc_kernel


@functools.lru_cache(maxsize=None)
def _build_layernorm(B, L):
    BB = 128
    R = BB * L // 2
    n_flat = B * L * HIDDEN

    def body(x_ref, gg_ref, bb_ref, o_ref):
        x = x_ref[...]
        half = lax.broadcasted_iota(jnp.int32, (2 * HIDDEN, 2 * HIDDEN), 0) // HIDDEN
        halfc = lax.broadcasted_iota(jnp.int32, (2 * HIDDEN, 2 * HIDDEN), 1) // HIDDEN
        S = jnp.where(half == halfc, 1.0 / HIDDEN, 0.0).astype(jnp.float32)
        mean = jnp.dot(x, S, preferred_element_type=jnp.float32)
        meansq = jnp.dot(x * x, S, preferred_element_type=jnp.float32)
        var = meansq - mean * mean
        y = (x - mean) * lax.rsqrt(var + EPS) * gg_ref[...] + bb_ref[...]
        o_ref[...] = y.reshape(BB, L // 2, 2 * HIDDEN)

    return pl.pallas_call(
        body,
        grid=(B // BB,),
        in_specs=[
            pl.BlockSpec((R, 2 * HIDDEN), lambda i: (i, 0)),
            pl.BlockSpec((1, 2 * HIDDEN), lambda i: (0, 0)),
            pl.BlockSpec((1, 2 * HIDDEN), lambda i: (0, 0)),
        ],
        out_specs=pl.BlockSpec((BB, L // 2, 2 * HIDDEN), lambda i: (i, 0, 0)),
        out_shape=jax.ShapeDtypeStruct((B, L // 2, 2 * HIDDEN), jnp.float32),
    )


NPART = 4


def kernel(diag_ids, med_ids, W_diag, W_med, gamma, beta):
    B, L = diag_ids.shape
    Bk = B // NPART
    nk = Bk * L
    g2 = jnp.concatenate([gamma, gamma]).reshape(1, 2 * HIDDEN)
    b2 = jnp.concatenate([beta, beta]).reshape(1, 2 * HIDDEN)
    d32 = diag_ids.astype(jnp.int32)
    m32 = med_ids.astype(jnp.int32)
    sc = _build_gather_sum(nk)
    tc = _build_layernorm(Bk, L)
    parts = []
    for k in range(NPART):
        dk, mk = lax.optimization_barrier(
            (lax.dynamic_slice_in_dim(d32, k * Bk, Bk),
             lax.dynamic_slice_in_dim(m32, k * Bk, Bk))
        )
        summed = sc(dk.reshape(nk), mk.reshape(nk), W_diag, W_med)
        part = tc(summed.reshape(nk // 2, 2 * HIDDEN), g2, b2)
        parts.append(part.reshape(Bk, L, HIDDEN))
    return jnp.concatenate(parts, axis=0)
